# baseline (speedup 1.0000x reference)
import functools

import jax
import jax.numpy as jnp
from jax import lax
from jax.experimental import pallas as pl
from jax.experimental.pallas import tpu as pltpu
from jax.experimental.pallas import tpu_sc as plsc

_PAIRS_A = (0, 2, 4, 6, 8, 10, 12, 14, 16, 18, 20, 22, 24, 0, 1, 4)
_PAIRS_B = (1, 3, 5, 7, 9, 11, 13, 15, 17, 19, 21, 23, 25, 2, 3, 6)

_I = 1000
_D = 26
_B = 16384
_P = 16

_JT = 8

_NC = 2
_NS = 16
_NW = _NC * _NS
_BW = _B // _NW
_GROUPS = _BW // 16
_QUARTERS = _BW // 128
_NROW = _P * _QUARTERS


def _rne_bf16_bits(v):
    u = lax.bitcast_convert_type(v, jnp.uint32)
    return (u + 0x7FFF + ((u >> 16) & 1)) >> 16


def _untile_body(x1_ref, x2_ref, o_ref):
    for jt in range(_JT):
        w = 128 if jt < _JT - 1 else _I - (_JT - 1) * 128
        a = _rne_bf16_bits(x1_ref[0, :, pl.ds(jt * 128, w)])
        b = _rne_bf16_bits(x2_ref[0, :, pl.ds(jt * 128, w)])
        o_ref[pl.ds(jt * _I, _I), pl.ds(0, w)] = lax.bitcast_convert_type(
            a | (b << 16), jnp.int32
        )


_G_SPLIT = 6


def _make_untile(g0, n):
    return pl.pallas_call(
        _untile_body,
        grid=(n,),
        in_specs=[
            pl.BlockSpec((1, _I, _I), lambda g, g0=g0: (g0 + g, 0, 0)),
            pl.BlockSpec((1, _I, _I), lambda g, g0=g0: (g0 + g + _P // 2, 0, 0)),
        ],
        out_specs=pl.BlockSpec((_JT * _I, 128), lambda g: (g, 0)),
        out_shape=jax.ShapeDtypeStruct((n * _JT * _I, 128), jnp.int32),
    )


_untile_h0 = _make_untile(0, _G_SPLIT)
_untile_h1 = _make_untile(_G_SPLIT, _P // 2 - _G_SPLIT)


def _sc1_body(xT_hbm, t0_hbm, singles_hbm, pidx_hbm, part_hbm,
              xT_v, t0_v, singles_v, pidx_v, out_v):
    wid = lax.axis_index("s") * _NC + lax.axis_index("c")
    base = wid * _BW

    pltpu.sync_copy(xT_hbm.at[:, pl.ds(base, _BW)], xT_v)
    pltpu.sync_copy(t0_hbm, t0_v)
    pltpu.sync_copy(singles_hbm, singles_v)

    for p in range(_P):
        ra, rb = _PAIRS_A[p], _PAIRS_B[p]
        for q in range(_QUARTERS):
            row = p * _QUARTERS + q

            def build(c, _, row=row, ra=ra, rb=rb, q=q, p=p):
                b0 = q * 128 + c * 16
                ia = xT_v[ra, pl.ds(b0, 16)]
                ib = xT_v[rb, pl.ds(b0, 16)]
                g = p % (_P // 2)
                g_local = g if g < _G_SPLIT else g - _G_SPLIT
                pidx_v[row, pl.ds(c * 16, 16)] = (
                    (g_local * _JT + (ib >> 7)) * (_I * 128)
                    + ia * 128
                    + (ib & 127)
                )
                return 0

            lax.fori_loop(0, 8, build, 0)

    def singles_acc(g, _):
        b0 = g * 16
        acc = t0_v[...]
        for d in range(_D):
            xv = xT_v[d, pl.ds(b0, 16)]
            acc = acc + plsc.load_gather(singles_v, [xv + d * _I])
        out_v[pl.ds(b0, 16)] = acc
        return 0

    lax.fori_loop(0, _GROUPS, singles_acc, 0)

    pltpu.sync_copy(pidx_v, pidx_hbm.at[wid])
    pltpu.sync_copy(out_v, part_hbm.at[pl.ds(base, _BW)])


_sc1_call = functools.partial(
    pl.kernel,
    mesh=plsc.VectorSubcoreMesh(core_axis_name="c", subcore_axis_name="s"),
    out_type=(
        jax.ShapeDtypeStruct((_NW, _NROW, 128), jnp.int32),
        jax.ShapeDtypeStruct((_B,), jnp.float32),
    ),
    compiler_params=pltpu.CompilerParams(needs_layout_passes=False),
    scratch_types=[
        pltpu.VMEM((_D, _BW), jnp.int32),
        pltpu.VMEM((16,), jnp.float32),
        pltpu.VMEM((_D * _I,), jnp.float32),
        pltpu.VMEM((_NROW, 128), jnp.int32),
        pltpu.VMEM((_BW,), jnp.float32),
    ],
)(_sc1_body)


def _make_sc2_body(g0, g1):
    n = 4 * (g1 - g0)
    off_lo = 4 * g0
    off_hi = 32 + 4 * g0

    def _sc2_body(pairs_hbm, pidx_hbm, part_hbm, out_hbm,
                  pidx_v, prow_v, out_v, sem):
        wid = lax.axis_index("s") * _NC + lax.axis_index("c")
        base = wid * _BW

        pltpu.sync_copy(pidx_hbm.at[wid], pidx_v)

        def fire_lo(t, _):
            pltpu.make_async_copy(
                pairs_hbm.at[pidx_v.at[off_lo + t]], prow_v.at[t], sem
            ).start()
            return 0

        def fire_hi(t, _):
            pltpu.make_async_copy(
                pairs_hbm.at[pidx_v.at[off_hi + t]], prow_v.at[n + t], sem
            ).start()
            return 0

        lax.fori_loop(0, n, fire_lo, 0)
        lax.fori_loop(0, n, fire_hi, 0)

        pltpu.sync_copy(part_hbm.at[pl.ds(base, _BW)], out_v)

        def drain_lo(t, _):
            pltpu.make_async_copy(
                pairs_hbm.at[pidx_v.at[off_lo + t]], prow_v.at[t], sem
            ).wait()
            return 0

        def drain_hi(t, _):
            pltpu.make_async_copy(
                pairs_hbm.at[pidx_v.at[off_hi + t]], prow_v.at[n + t], sem
            ).wait()
            return 0

        lax.fori_loop(0, n, drain_lo, 0)
        lax.fori_loop(0, n, drain_hi, 0)

        mask_hi = jnp.full((16,), -65536, jnp.int32)
        for local in range(2 * n):
            jg = (off_lo + local) if local < n else (off_hi + local - n)
            q = jg % _QUARTERS
            high = local >= n

            def pairs_acc(c, _, local=local, q=q, high=high):
                sl = pl.ds(q * 128 + c * 16, 16)
                u = prow_v[local, pl.ds(c * 16, 16)]
                if high:
                    f = plsc.bitcast(u & mask_hi, jnp.float32)
                else:
                    f = plsc.bitcast(u << 16, jnp.float32)
                out_v[sl] = out_v[sl] + f
                return 0

            lax.fori_loop(0, 8, pairs_acc, 0)

        pltpu.sync_copy(out_v, out_hbm.at[pl.ds(base, _BW)])

    return _sc2_body


def _make_sc2_call(g0, g1):
    return functools.partial(
        pl.kernel,
        mesh=plsc.VectorSubcoreMesh(core_axis_name="c", subcore_axis_name="s"),
        out_type=jax.ShapeDtypeStruct((_B,), jnp.float32),
        compiler_params=pltpu.CompilerParams(needs_layout_passes=False),
        scratch_types=[
            pltpu.VMEM((_NROW, 128), jnp.int32),
            pltpu.VMEM((8 * (g1 - g0), 128), jnp.int32),
            pltpu.VMEM((_BW,), jnp.float32),
            pltpu.SemaphoreType.DMA,
        ],
    )(_make_sc2_body(g0, g1))


_sc2_h0 = _make_sc2_call(0, _G_SPLIT)
_sc2_h1 = _make_sc2_call(_G_SPLIT, _P // 2)


@jax.jit
def kernel(x, theta0, theta_singles, theta_pairs):
    xT = x.T.astype(jnp.int32)
    t0 = jnp.broadcast_to(theta0.astype(jnp.float32).reshape(1), (16,))
    singles = theta_singles.reshape(-1).astype(jnp.float32)
    tp = theta_pairs.astype(jnp.float32)
    half0 = _untile_h0(tp, tp).reshape(-1)
    half1 = _untile_h1(tp, tp).reshape(-1)
    pidx, part = _sc1_call(xT, t0, singles)
    part2 = _sc2_h0(half0, pidx, part)
    return _sc2_h1(half1, pidx, part2)


# speedup vs baseline: 1.0594x; 1.0594x over previous
import functools

import jax
import jax.numpy as jnp
from jax import lax
from jax.experimental import pallas as pl
from jax.experimental.pallas import tpu as pltpu
from jax.experimental.pallas import tpu_sc as plsc

_PAIRS_A = (0, 2, 4, 6, 8, 10, 12, 14, 16, 18, 20, 22, 24, 0, 1, 4)
_PAIRS_B = (1, 3, 5, 7, 9, 11, 13, 15, 17, 19, 21, 23, 25, 2, 3, 6)

_I = 1000
_D = 26
_B = 16384
_P = 16

_JT = 8

_NC = 2
_NS = 16
_NW = _NC * _NS
_BW = _B // _NW
_GROUPS = _BW // 16
_QUARTERS = _BW // 128
_NROW = _P * _QUARTERS


def _rne_bf16_bits(v):
    u = lax.bitcast_convert_type(v, jnp.uint32)
    return (u + 0x7FFF + ((u >> 16) & 1)) >> 16


def _untile_body(x1_ref, x2_ref, o_ref):
    for jt in range(_JT):
        w = 128 if jt < _JT - 1 else _I - (_JT - 1) * 128
        a = _rne_bf16_bits(x1_ref[0, :, pl.ds(jt * 128, w)])
        b = _rne_bf16_bits(x2_ref[0, :, pl.ds(jt * 128, w)])
        o_ref[pl.ds(jt * _I, _I), pl.ds(0, w)] = lax.bitcast_convert_type(
            a | (b << 16), jnp.int32
        )


_G_SPLIT = _P // 2


def _make_untile(g0, n):
    return pl.pallas_call(
        _untile_body,
        grid=(n,),
        in_specs=[
            pl.BlockSpec((1, _I, _I), lambda g, g0=g0: (g0 + g, 0, 0)),
            pl.BlockSpec((1, _I, _I), lambda g, g0=g0: (g0 + g + _P // 2, 0, 0)),
        ],
        out_specs=pl.BlockSpec((_JT * _I, 128), lambda g: (g, 0)),
        out_shape=jax.ShapeDtypeStruct((n * _JT * _I, 128), jnp.int32),
    )


_untile_h0 = _make_untile(0, _G_SPLIT)


def _sc1_body(xT_hbm, t0_hbm, singles_hbm, pidx_hbm, part_hbm,
              xT_v, t0_v, singles_v, pidx_v, out_v):
    wid = lax.axis_index("s") * _NC + lax.axis_index("c")
    base = wid * _BW

    pltpu.sync_copy(xT_hbm.at[:, pl.ds(base, _BW)], xT_v)
    pltpu.sync_copy(t0_hbm, t0_v)
    pltpu.sync_copy(singles_hbm, singles_v)

    for p in range(_P):
        ra, rb = _PAIRS_A[p], _PAIRS_B[p]
        for q in range(_QUARTERS):
            row = p * _QUARTERS + q

            def build(c, _, row=row, ra=ra, rb=rb, q=q, p=p):
                b0 = q * 128 + c * 16
                ia = xT_v[ra, pl.ds(b0, 16)]
                ib = xT_v[rb, pl.ds(b0, 16)]
                g = p % (_P // 2)
                g_local = g if g < _G_SPLIT else g - _G_SPLIT
                pidx_v[row, pl.ds(c * 16, 16)] = (
                    (g_local * _JT + (ib >> 7)) * (_I * 128)
                    + ia * 128
                    + (ib & 127)
                )
                return 0

            lax.fori_loop(0, 8, build, 0)

    def singles_acc(g, _):
        b0 = g * 16
        acc = t0_v[...]
        for d in range(_D):
            xv = xT_v[d, pl.ds(b0, 16)]
            acc = acc + plsc.load_gather(singles_v, [xv + d * _I])
        out_v[pl.ds(b0, 16)] = acc
        return 0

    lax.fori_loop(0, _GROUPS, singles_acc, 0)

    pltpu.sync_copy(pidx_v, pidx_hbm.at[wid])
    pltpu.sync_copy(out_v, part_hbm.at[pl.ds(base, _BW)])


_sc1_call = functools.partial(
    pl.kernel,
    mesh=plsc.VectorSubcoreMesh(core_axis_name="c", subcore_axis_name="s"),
    out_type=(
        jax.ShapeDtypeStruct((_NW, _NROW, 128), jnp.int32),
        jax.ShapeDtypeStruct((_B,), jnp.float32),
    ),
    compiler_params=pltpu.CompilerParams(needs_layout_passes=False),
    scratch_types=[
        pltpu.VMEM((_D, _BW), jnp.int32),
        pltpu.VMEM((16,), jnp.float32),
        pltpu.VMEM((_D * _I,), jnp.float32),
        pltpu.VMEM((_NROW, 128), jnp.int32),
        pltpu.VMEM((_BW,), jnp.float32),
    ],
)(_sc1_body)


def _make_sc2_body(g0, g1):
    n = 4 * (g1 - g0)
    off_lo = 4 * g0
    off_hi = 32 + 4 * g0

    def _sc2_body(pairs_hbm, pidx_hbm, part_hbm, out_hbm,
                  pidx_v, prow_v, out_v, sem):
        wid = lax.axis_index("s") * _NC + lax.axis_index("c")
        base = wid * _BW

        pltpu.sync_copy(pidx_hbm.at[wid], pidx_v)

        def fire_lo(t, _):
            pltpu.make_async_copy(
                pairs_hbm.at[pidx_v.at[off_lo + t]], prow_v.at[t], sem
            ).start()
            return 0

        def fire_hi(t, _):
            pltpu.make_async_copy(
                pairs_hbm.at[pidx_v.at[off_hi + t]], prow_v.at[n + t], sem
            ).start()
            return 0

        lax.fori_loop(0, n, fire_lo, 0)
        lax.fori_loop(0, n, fire_hi, 0)

        pltpu.sync_copy(part_hbm.at[pl.ds(base, _BW)], out_v)

        def drain_lo(t, _):
            pltpu.make_async_copy(
                pairs_hbm.at[pidx_v.at[off_lo + t]], prow_v.at[t], sem
            ).wait()
            return 0

        def drain_hi(t, _):
            pltpu.make_async_copy(
                pairs_hbm.at[pidx_v.at[off_hi + t]], prow_v.at[n + t], sem
            ).wait()
            return 0

        lax.fori_loop(0, n, drain_lo, 0)
        lax.fori_loop(0, n, drain_hi, 0)

        mask_hi = jnp.full((16,), -65536, jnp.int32)
        for local in range(2 * n):
            jg = (off_lo + local) if local < n else (off_hi + local - n)
            q = jg % _QUARTERS
            high = local >= n

            def pairs_acc(c, _, local=local, q=q, high=high):
                sl = pl.ds(q * 128 + c * 16, 16)
                u = prow_v[local, pl.ds(c * 16, 16)]
                if high:
                    f = plsc.bitcast(u & mask_hi, jnp.float32)
                else:
                    f = plsc.bitcast(u << 16, jnp.float32)
                out_v[sl] = out_v[sl] + f
                return 0

            lax.fori_loop(0, 8, pairs_acc, 0)

        pltpu.sync_copy(out_v, out_hbm.at[pl.ds(base, _BW)])

    return _sc2_body


def _make_sc2_call(g0, g1):
    return functools.partial(
        pl.kernel,
        mesh=plsc.VectorSubcoreMesh(core_axis_name="c", subcore_axis_name="s"),
        out_type=jax.ShapeDtypeStruct((_B,), jnp.float32),
        compiler_params=pltpu.CompilerParams(needs_layout_passes=False),
        scratch_types=[
            pltpu.VMEM((_NROW, 128), jnp.int32),
            pltpu.VMEM((8 * (g1 - g0), 128), jnp.int32),
            pltpu.VMEM((_BW,), jnp.float32),
            pltpu.SemaphoreType.DMA,
        ],
    )(_make_sc2_body(g0, g1))


_sc2_h0 = _make_sc2_call(0, _G_SPLIT)


@jax.jit
def kernel(x, theta0, theta_singles, theta_pairs):
    xT = x.T.astype(jnp.int32)
    t0 = jnp.broadcast_to(theta0.astype(jnp.float32).reshape(1), (16,))
    singles = theta_singles.reshape(-1).astype(jnp.float32)
    tp = theta_pairs.astype(jnp.float32)
    pairs_lin = _untile_h0(tp, tp).reshape(-1)
    pidx, part = _sc1_call(xT, t0, singles)
    return _sc2_h0(pairs_lin, pidx, part)


# speedup vs baseline: 1.0829x; 1.0222x over previous
import functools

import jax
import jax.numpy as jnp
from jax import lax
from jax.experimental import pallas as pl
from jax.experimental.pallas import tpu as pltpu
from jax.experimental.pallas import tpu_sc as plsc

_PAIRS_A = (0, 2, 4, 6, 8, 10, 12, 14, 16, 18, 20, 22, 24, 0, 1, 4)
_PAIRS_B = (1, 3, 5, 7, 9, 11, 13, 15, 17, 19, 21, 23, 25, 2, 3, 6)

_I = 1000
_D = 26
_B = 16384
_P = 16

_JT = 8

_NC = 2
_NS = 16
_NW = _NC * _NS
_BW = _B // _NW
_GROUPS = _BW // 16
_QUARTERS = _BW // 128
_NROW = _P * _QUARTERS


def _rne_bf16_bits(v):
    u = lax.bitcast_convert_type(v, jnp.uint32)
    return (u + 0x7FFF + ((u >> 16) & 1)) >> 16


def _untile_body(x1_ref, x2_ref, o_ref):
    for jt in range(_JT):
        w = 128 if jt < _JT - 1 else _I - (_JT - 1) * 128
        a = _rne_bf16_bits(x1_ref[0, :, pl.ds(jt * 128, w)])
        b = _rne_bf16_bits(x2_ref[0, :, pl.ds(jt * 128, w)])
        o_ref[pl.ds(jt * _I, _I), pl.ds(0, w)] = lax.bitcast_convert_type(
            a | (b << 16), jnp.int32
        )


_G_SPLIT = _P // 2


def _make_untile(g0, n):
    return pl.pallas_call(
        _untile_body,
        grid=(n,),
        in_specs=[
            pl.BlockSpec((1, _I, _I), lambda g, g0=g0: (g0 + g, 0, 0)),
            pl.BlockSpec((1, _I, _I), lambda g, g0=g0: (g0 + g + _P // 2, 0, 0)),
        ],
        out_specs=pl.BlockSpec((_JT * _I, 128), lambda g: (g, 0)),
        out_shape=jax.ShapeDtypeStruct((n * _JT * _I, 128), jnp.int32),
    )


_untile_h0 = _make_untile(0, _G_SPLIT)


def _sc1_body(xT_hbm, t0_hbm, singles_hbm, pidx_hbm, part_hbm,
              xT_v, t0_v, singles_v, pidx_v, out_v):
    wid = lax.axis_index("s") * _NC + lax.axis_index("c")
    base = wid * _BW

    pltpu.sync_copy(xT_hbm.at[:, pl.ds(base, _BW)], xT_v)
    pltpu.sync_copy(t0_hbm, t0_v.at[pl.ds(0, 1)])
    pltpu.sync_copy(singles_hbm, singles_v)
    t0vec = jnp.full((16,), t0_v[...][0], jnp.float32)

    for p in range(_P):
        ra, rb = _PAIRS_A[p], _PAIRS_B[p]
        for q in range(_QUARTERS):
            row = p * _QUARTERS + q

            def build(c, _, row=row, ra=ra, rb=rb, q=q, p=p):
                b0 = q * 128 + c * 16
                ia = xT_v[ra, pl.ds(b0, 16)]
                ib = xT_v[rb, pl.ds(b0, 16)]
                g = p % (_P // 2)
                g_local = g if g < _G_SPLIT else g - _G_SPLIT
                pidx_v[row, pl.ds(c * 16, 16)] = (
                    (g_local * _JT + (ib >> 7)) * (_I * 128)
                    + ia * 128
                    + (ib & 127)
                )
                return 0

            lax.fori_loop(0, 8, build, 0)

    def singles_acc(g, _):
        b0 = g * 16
        acc = t0vec
        for d in range(_D):
            xv = xT_v[d, pl.ds(b0, 16)]
            acc = acc + plsc.load_gather(
                singles_v, [jnp.full((16,), d, jnp.int32), xv]
            )
        out_v[pl.ds(b0, 16)] = acc
        return 0

    lax.fori_loop(0, _GROUPS, singles_acc, 0)

    pltpu.sync_copy(pidx_v, pidx_hbm.at[wid])
    pltpu.sync_copy(out_v, part_hbm.at[pl.ds(base, _BW)])


_sc1_call = functools.partial(
    pl.kernel,
    mesh=plsc.VectorSubcoreMesh(core_axis_name="c", subcore_axis_name="s"),
    out_type=(
        jax.ShapeDtypeStruct((_NW, _NROW, 128), jnp.int32),
        jax.ShapeDtypeStruct((_B,), jnp.float32),
    ),
    compiler_params=pltpu.CompilerParams(needs_layout_passes=False),
    scratch_types=[
        pltpu.VMEM((_D, _BW), jnp.int32),
        pltpu.VMEM((16,), jnp.float32),
        pltpu.VMEM((_D, _I), jnp.float32),
        pltpu.VMEM((_NROW, 128), jnp.int32),
        pltpu.VMEM((_BW,), jnp.float32),
    ],
)(_sc1_body)


def _make_sc2_body(g0, g1):
    n = 4 * (g1 - g0)
    off_lo = 4 * g0
    off_hi = 32 + 4 * g0

    def _sc2_body(pairs_hbm, pidx_hbm, part_hbm, out_hbm,
                  pidx_v, prow_v, out_v, sem):
        wid = lax.axis_index("s") * _NC + lax.axis_index("c")
        base = wid * _BW

        pltpu.sync_copy(pidx_hbm.at[wid], pidx_v)

        def fire_lo(t, _):
            pltpu.make_async_copy(
                pairs_hbm.at[pidx_v.at[off_lo + t]], prow_v.at[t], sem
            ).start()
            return 0

        def fire_hi(t, _):
            pltpu.make_async_copy(
                pairs_hbm.at[pidx_v.at[off_hi + t]], prow_v.at[n + t], sem
            ).start()
            return 0

        lax.fori_loop(0, n, fire_lo, 0)
        lax.fori_loop(0, n, fire_hi, 0)

        pltpu.sync_copy(part_hbm.at[pl.ds(base, _BW)], out_v)

        def drain_lo(t, _):
            pltpu.make_async_copy(
                pairs_hbm.at[pidx_v.at[off_lo + t]], prow_v.at[t], sem
            ).wait()
            return 0

        def drain_hi(t, _):
            pltpu.make_async_copy(
                pairs_hbm.at[pidx_v.at[off_hi + t]], prow_v.at[n + t], sem
            ).wait()
            return 0

        lax.fori_loop(0, n, drain_lo, 0)
        lax.fori_loop(0, n, drain_hi, 0)

        mask_hi = jnp.full((16,), -65536, jnp.int32)
        for local in range(2 * n):
            jg = (off_lo + local) if local < n else (off_hi + local - n)
            q = jg % _QUARTERS
            high = local >= n

            def pairs_acc(c, _, local=local, q=q, high=high):
                sl = pl.ds(q * 128 + c * 16, 16)
                u = prow_v[local, pl.ds(c * 16, 16)]
                if high:
                    f = plsc.bitcast(u & mask_hi, jnp.float32)
                else:
                    f = plsc.bitcast(u << 16, jnp.float32)
                out_v[sl] = out_v[sl] + f
                return 0

            lax.fori_loop(0, 8, pairs_acc, 0)

        pltpu.sync_copy(out_v, out_hbm.at[pl.ds(base, _BW)])

    return _sc2_body


def _make_sc2_call(g0, g1):
    return functools.partial(
        pl.kernel,
        mesh=plsc.VectorSubcoreMesh(core_axis_name="c", subcore_axis_name="s"),
        out_type=jax.ShapeDtypeStruct((_B,), jnp.float32),
        compiler_params=pltpu.CompilerParams(needs_layout_passes=False),
        scratch_types=[
            pltpu.VMEM((_NROW, 128), jnp.int32),
            pltpu.VMEM((8 * (g1 - g0), 128), jnp.int32),
            pltpu.VMEM((_BW,), jnp.float32),
            pltpu.SemaphoreType.DMA,
        ],
    )(_make_sc2_body(g0, g1))


_sc2_h0 = _make_sc2_call(0, _G_SPLIT)


@jax.jit
def kernel(x, theta0, theta_singles, theta_pairs):
    xT = x.T.astype(jnp.int32)
    t0 = theta0.astype(jnp.float32).reshape(1)
    singles = theta_singles.astype(jnp.float32)
    tp = theta_pairs.astype(jnp.float32)
    pairs_lin = _untile_h0(tp, tp).reshape(-1)
    pidx, part = _sc1_call(xT, t0, singles)
    return _sc2_h0(pairs_lin, pidx, part)
